# baseline (speedup 1.0000x reference)
import functools

import jax
import jax.numpy as jnp
from jax import lax
from jax.experimental import pallas as pl
from jax.experimental.pallas import tpu as pltpu
from jax.experimental.pallas import tpu_sc as plsc

_LANES = 16
_CHUNK = 8
_NBUF = 3


def _sc_part(nbr, att, bias, cut):
    _, K = att.shape
    D = nbr.shape[1]
    n_chunks = cut // _CHUNK
    num_workers = 32
    n_dblk = D // _LANES

    mesh = plsc.VectorSubcoreMesh(core_axis_name="c", subcore_axis_name="s")

    @functools.partial(
        pl.kernel,
        mesh=mesh,
        out_type=jax.ShapeDtypeStruct((cut, D), jnp.float32),
        scratch_types=[
            pltpu.VMEM((_CHUNK * K, D), jnp.float32),
            pltpu.VMEM((_CHUNK * K, D), jnp.float32),
            pltpu.VMEM((_CHUNK * K, D), jnp.float32),
            pltpu.VMEM((_CHUNK, K), jnp.float32),
            pltpu.VMEM((_CHUNK, K), jnp.float32),
            pltpu.VMEM((_CHUNK, K), jnp.float32),
            pltpu.VMEM((_CHUNK, D), jnp.float32),
            pltpu.VMEM((_CHUNK, D), jnp.float32),
            pltpu.VMEM((_CHUNK, D), jnp.float32),
            pltpu.VMEM((D,), jnp.float32),
        ] + [pltpu.SemaphoreType.DMA] * 9,
    )
    def sc_kernel(nbr_hbm, att_hbm, bias_hbm, out_hbm,
                  nbr_v0, nbr_v1, nbr_v2, att_v0, att_v1, att_v2,
                  out_v0, out_v1, out_v2, bias_v,
                  sn0, sn1, sn2, sa0, sa1, sa2, so0, so1, so2):
        nbr_bufs = (nbr_v0, nbr_v1, nbr_v2)
        att_bufs = (att_v0, att_v1, att_v2)
        out_bufs = (out_v0, out_v1, out_v2)
        sems_n = (sn0, sn1, sn2)
        sems_a = (sa0, sa1, sa2)
        sems_o = (so0, so1, so2)
        cid = lax.axis_index("c")
        sid = lax.axis_index("s")
        wid = sid * 2 + cid
        pltpu.sync_copy(bias_hbm, bias_v)
        n_my = (n_chunks - wid + num_workers - 1) // num_workers

        def chunk_base(t):
            return (wid + t * num_workers) * _CHUNK

        def issue(t, b):
            base = chunk_base(t)
            pltpu.async_copy(nbr_hbm.at[pl.ds(base * K, _CHUNK * K), :],
                             nbr_bufs[b], sems_n[b])
            pltpu.async_copy(att_hbm.at[pl.ds(base, _CHUNK), :],
                             att_bufs[b], sems_a[b])

        def drain(t, b):
            base = chunk_base(t)
            pltpu.make_async_copy(nbr_hbm.at[pl.ds(base * K, _CHUNK * K), :],
                                  nbr_bufs[b], sems_n[b]).wait()
            pltpu.make_async_copy(att_hbm.at[pl.ds(base, _CHUNK), :],
                                  att_bufs[b], sems_a[b]).wait()

        def store_wait(t, b):
            pltpu.make_async_copy(out_bufs[b],
                                  out_hbm.at[pl.ds(chunk_base(t), _CHUNK), :],
                                  sems_o[b]).wait()

        def compute(t, b):
            nv = nbr_bufs[b]
            av = att_bufs[b]
            ov = out_bufs[b]

            @pl.when(t >= _NBUF)
            def _():
                store_wait(t - _NBUF, b)

            def node_body(i, c):
                krow = i * K
                accs = [bias_v[pl.ds(j * _LANES, _LANES)] for j in range(n_dblk)]
                att_rows = [
                    av[i, pl.ds(kk * _LANES, _LANES)]
                    for kk in range(K // _LANES)
                ]
                for k in range(K):
                    a = att_rows[k // _LANES][k % _LANES]
                    row = krow + k
                    for j in range(n_dblk):
                        accs[j] = accs[j] + a * nv[row, pl.ds(j * _LANES, _LANES)]
                for j in range(n_dblk):
                    ov[i, pl.ds(j * _LANES, _LANES)] = accs[j]
                return c

            lax.fori_loop(0, _CHUNK, node_body, 0)
            pltpu.async_copy(ov, out_hbm.at[pl.ds(chunk_base(t), _CHUNK), :],
                             sems_o[b])

        issue(0, 0)

        @pl.when(1 < n_my)
        def _():
            issue(1, 1)

        def outer(it, carry):
            t0 = it * _NBUF
            for b in range(_NBUF):
                t = t0 + b

                @pl.when(t + 2 < n_my)
                def _():
                    issue(t + 2, (b + 2) % _NBUF)

                @pl.when(t < n_my)
                def _():
                    drain(t, b)
                    compute(t, b)

            return carry

        lax.fori_loop(0, (n_my + _NBUF - 1) // _NBUF, outer, 0)

        for b in range(_NBUF):
            @pl.when(n_my > b)
            def _(b=b):
                t_last = ((n_my - 1 - b) // _NBUF) * _NBUF + b
                store_wait(t_last, b)

    return sc_kernel(nbr, att, bias)


def _tc_part(nbr, att2, bias, cut, block):
    NK, D = nbr.shape
    K = att2.shape[1]
    N = NK // K
    M = N - cut
    assert M % block == 0 and cut % block == 0
    grid = M // block
    off = cut // block

    def body(nbr_ref, att_ref, bias_ref, out_ref):
        v = nbr_ref[...].reshape(block, K, D)
        at = att_ref[...]
        acc = jnp.sum(v * at[:, :, None], axis=1)
        out_ref[...] = acc + jnp.broadcast_to(bias_ref[...], (block, D))

    return pl.pallas_call(
        body,
        grid=(grid,),
        in_specs=[
            pl.BlockSpec((block * K, D), lambda i: (i + off, 0)),
            pl.BlockSpec((block, K), lambda i: (i + off, 0)),
            pl.BlockSpec((1, D), lambda i: (0, 0)),
        ],
        out_specs=pl.BlockSpec((block, D), lambda i: (i, 0)),
        out_shape=jax.ShapeDtypeStruct((M, D), jnp.float32),
    )(nbr, att2, bias.reshape(1, D))


def kernel(nodes, neighbors, attention_scores, bias):
    del nodes
    N, K, _, D = neighbors.shape
    att = attention_scores.reshape(N, K)
    cut = 5200
    nbr = neighbors.reshape(N * K, D)
    out_sc = _sc_part(nbr, att, bias, cut)
    out_tc = _tc_part(nbr, att, bias, cut, block=400)
    return jnp.concatenate([out_sc, out_tc], axis=0)


# speedup vs baseline: 1.0072x; 1.0072x over previous
import functools

import jax
import jax.numpy as jnp
from jax import lax
from jax.experimental import pallas as pl
from jax.experimental.pallas import tpu as pltpu
from jax.experimental.pallas import tpu_sc as plsc

_LANES = 16
_CHUNK = 8
_NBUF = 3


def _sc_part(nbr, att, bias, cut):
    _, K = att.shape
    D = nbr.shape[1]
    n_chunks = cut // _CHUNK
    num_workers = 32
    n_dblk = D // _LANES

    mesh = plsc.VectorSubcoreMesh(core_axis_name="c", subcore_axis_name="s")

    @functools.partial(
        pl.kernel,
        mesh=mesh,
        out_type=jax.ShapeDtypeStruct((cut, D), jnp.float32),
        scratch_types=[
            pltpu.VMEM((_CHUNK * K, D), jnp.float32),
            pltpu.VMEM((_CHUNK * K, D), jnp.float32),
            pltpu.VMEM((_CHUNK * K, D), jnp.float32),
            pltpu.VMEM((_CHUNK, K), jnp.float32),
            pltpu.VMEM((_CHUNK, K), jnp.float32),
            pltpu.VMEM((_CHUNK, K), jnp.float32),
            pltpu.VMEM((_CHUNK, D), jnp.float32),
            pltpu.VMEM((_CHUNK, D), jnp.float32),
            pltpu.VMEM((_CHUNK, D), jnp.float32),
            pltpu.VMEM((D,), jnp.float32),
        ] + [pltpu.SemaphoreType.DMA] * 9,
    )
    def sc_kernel(nbr_hbm, att_hbm, bias_hbm, out_hbm,
                  nbr_v0, nbr_v1, nbr_v2, att_v0, att_v1, att_v2,
                  out_v0, out_v1, out_v2, bias_v,
                  sn0, sn1, sn2, sa0, sa1, sa2, so0, so1, so2):
        nbr_bufs = (nbr_v0, nbr_v1, nbr_v2)
        att_bufs = (att_v0, att_v1, att_v2)
        out_bufs = (out_v0, out_v1, out_v2)
        sems_n = (sn0, sn1, sn2)
        sems_a = (sa0, sa1, sa2)
        sems_o = (so0, so1, so2)
        cid = lax.axis_index("c")
        sid = lax.axis_index("s")
        wid = sid * 2 + cid
        pltpu.sync_copy(bias_hbm, bias_v)
        n_my = (n_chunks - wid + num_workers - 1) // num_workers

        def chunk_base(t):
            return (wid + t * num_workers) * _CHUNK

        def issue(t, b):
            base = chunk_base(t)
            pltpu.async_copy(nbr_hbm.at[pl.ds(base * K, _CHUNK * K), :],
                             nbr_bufs[b], sems_n[b])
            pltpu.async_copy(att_hbm.at[pl.ds(base, _CHUNK), :],
                             att_bufs[b], sems_a[b])

        def drain(t, b):
            base = chunk_base(t)
            pltpu.make_async_copy(nbr_hbm.at[pl.ds(base * K, _CHUNK * K), :],
                                  nbr_bufs[b], sems_n[b]).wait()
            pltpu.make_async_copy(att_hbm.at[pl.ds(base, _CHUNK), :],
                                  att_bufs[b], sems_a[b]).wait()

        def store_wait(t, b):
            pltpu.make_async_copy(out_bufs[b],
                                  out_hbm.at[pl.ds(chunk_base(t), _CHUNK), :],
                                  sems_o[b]).wait()

        def compute(t, b):
            nv = nbr_bufs[b]
            av = att_bufs[b]
            ov = out_bufs[b]

            @pl.when(t >= _NBUF)
            def _():
                store_wait(t - _NBUF, b)

            def node_body(i, c):
                krow = i * K
                accs = [bias_v[pl.ds(j * _LANES, _LANES)] for j in range(n_dblk)]
                att_rows = [
                    av[i, pl.ds(kk * _LANES, _LANES)]
                    for kk in range(K // _LANES)
                ]
                for k in range(K):
                    a = att_rows[k // _LANES][k % _LANES]
                    row = krow + k
                    for j in range(n_dblk):
                        accs[j] = accs[j] + a * nv[row, pl.ds(j * _LANES, _LANES)]
                for j in range(n_dblk):
                    ov[i, pl.ds(j * _LANES, _LANES)] = accs[j]
                return c

            lax.fori_loop(0, _CHUNK, node_body, 0)
            pltpu.async_copy(ov, out_hbm.at[pl.ds(chunk_base(t), _CHUNK), :],
                             sems_o[b])

        issue(0, 0)

        @pl.when(1 < n_my)
        def _():
            issue(1, 1)

        def outer(it, carry):
            t0 = it * _NBUF
            for b in range(_NBUF):
                t = t0 + b

                @pl.when(t + 2 < n_my)
                def _():
                    issue(t + 2, (b + 2) % _NBUF)

                @pl.when(t < n_my)
                def _():
                    drain(t, b)
                    compute(t, b)

            return carry

        lax.fori_loop(0, (n_my + _NBUF - 1) // _NBUF, outer, 0)

        for b in range(_NBUF):
            @pl.when(n_my > b)
            def _(b=b):
                t_last = ((n_my - 1 - b) // _NBUF) * _NBUF + b
                store_wait(t_last, b)

    return sc_kernel(nbr, att, bias)


def _tc_part(nbr, att2, bias, cut, block):
    NK, D = nbr.shape
    K = att2.shape[1]
    N = NK // K
    M = N - cut
    assert M % block == 0 and cut % block == 0
    grid = M // block
    off = cut // block

    def body(nbr_ref, att_ref, bias_ref, out_ref):
        v = nbr_ref[...].reshape(block, K, D)
        at = att_ref[...]
        acc = jnp.sum(v * at[:, :, None], axis=1)
        out_ref[...] = acc + jnp.broadcast_to(bias_ref[...], (block, D))

    return pl.pallas_call(
        body,
        grid=(grid,),
        in_specs=[
            pl.BlockSpec((block * K, D), lambda i: (i + off, 0)),
            pl.BlockSpec((block, K), lambda i: (i + off, 0)),
            pl.BlockSpec((1, D), lambda i: (0, 0)),
        ],
        out_specs=pl.BlockSpec((block, D), lambda i: (i, 0)),
        out_shape=jax.ShapeDtypeStruct((M, D), jnp.float32),
    )(nbr, att2, bias.reshape(1, D))


def kernel(nodes, neighbors, attention_scores, bias):
    del nodes
    N, K, _, D = neighbors.shape
    att = attention_scores.reshape(N, K)
    cut = 5400
    nbr = neighbors.reshape(N * K, D)
    out_sc = _sc_part(nbr, att, bias, cut)
    out_tc = _tc_part(nbr, att, bias, cut, block=200)
    return jnp.concatenate([out_sc, out_tc], axis=0)
